# Initial kernel scaffold; baseline (speedup 1.0000x reference)
import jax
import jax.numpy as jnp
from jax.experimental import pallas as pl


def kernel(x, month_w, day_w, weekday_w, hour_w, minute_w):
    raise NotImplementedError("write your pallas kernel here")


# speedup vs baseline: 3.7392x; 3.7392x over previous
import functools

import jax
import jax.numpy as jnp
from jax import lax
from jax.experimental import pallas as pl
from jax.experimental.pallas import tpu as pltpu
from jax.experimental.pallas import tpu_sc as plsc

D = 1024
B, S, F = 4, 8192, 5
NC, NS, L = 2, 16, 16
NW = NC * NS
N = B * S
TPW = N // NW
CHUNK = 64
NCHUNKS = TPW // CHUNK
SPW = S // (NW // B)
WPB = NW // B

_MAXROW = (11, 30, 6, 23, 3)


def _lut_body(mo, dw, wd, hr, mi, lut):
    c = lax.broadcasted_iota(jnp.int32, (32, 1), 0)
    acc = jnp.broadcast_to(mo[0:1] + dw[0:1] + wd[0:1] + hr[0:1] + mi[0:1], (32, D))
    for f, (ref, r) in enumerate(zip((mo, dw, wd, hr, mi), _MAXROW)):
        bit = ((c >> f) & 1).astype(jnp.float32)
        acc = acc + bit * (ref[r:r + 1] - ref[0:1])
    lut[...] = acc


_build_lut = pl.pallas_call(
    _lut_body,
    out_shape=jax.ShapeDtypeStruct((32, D), jnp.float32),
)


def _sc_body(f0, f1, f2, f3, f4, lut_hbm, out_hbm, x0, x1, x2, x3, x4, codes_v, rows_v, sem):
    cid = lax.axis_index("c")
    sid = lax.axis_index("s")
    wid = sid * NC + cid
    b = wid // WPB
    off = (wid % WPB) * SPW

    xbufs = (x0, x1, x2, x3, x4)
    for f, fh in enumerate((f0, f1, f2, f3, f4)):
        pltpu.sync_copy(fh.at[pl.ds(wid * TPW, TPW)], xbufs[f])

    def cbody(j, carry):
        sl = pl.ds(j * L, L)
        code = jnp.zeros((L,), jnp.int32)
        for f in range(F):
            g = xbufs[f][sl]
            code = code | jnp.where(g >= 1, 1 << f, 0)
        codes_v[sl] = code
        return carry

    lax.fori_loop(0, TPW // L, cbody, 0)

    for i in range(NCHUNKS):
        cp = pltpu.async_copy(
            lut_hbm.at[codes_v.at[pl.ds(i * CHUNK, CHUNK)]], rows_v, sem)
        cp.wait()
        pltpu.sync_copy(rows_v, out_hbm.at[b, pl.ds(off + i * CHUNK, CHUNK), :])


@functools.cache
def _sc_gather():
    return pl.kernel(
        _sc_body,
        out_type=jax.ShapeDtypeStruct((B, S, D), jnp.float32),
        mesh=plsc.VectorSubcoreMesh(
            core_axis_name="c", subcore_axis_name="s",
            num_cores=NC, num_subcores=NS),
        scratch_types=[
            pltpu.VMEM((TPW,), jnp.int32),
            pltpu.VMEM((TPW,), jnp.int32),
            pltpu.VMEM((TPW,), jnp.int32),
            pltpu.VMEM((TPW,), jnp.int32),
            pltpu.VMEM((TPW,), jnp.int32),
            pltpu.VMEM((TPW,), jnp.int32),
            pltpu.VMEM((CHUNK, D), jnp.float32),
            pltpu.SemaphoreType.DMA,
        ],
    )


def kernel(x, month_w, day_w, weekday_w, hour_w, minute_w):
    lut = _build_lut(month_w, day_w, weekday_w, hour_w, minute_w)
    xi = x.astype(jnp.int32).reshape(N, F)
    feats = tuple(xi[:, f] for f in range(F))
    return _sc_gather()(*feats, lut)


# speedup vs baseline: 3.7625x; 1.0062x over previous
import functools

import jax
import jax.numpy as jnp
from jax import lax
from jax.experimental import pallas as pl
from jax.experimental.pallas import tpu as pltpu
from jax.experimental.pallas import tpu_sc as plsc

D = 1024
B, S, F = 4, 8192, 5
NC, NS, L = 2, 16, 16
NW = NC * NS
N = B * S
TPW = N // NW
CHUNK = 32
NCHUNKS = TPW // CHUNK
NBUF = 2
SPW = S // (NW // B)
WPB = NW // B

_MAXROW = (11, 30, 6, 23, 3)


def _lut_body(mo, dw, wd, hr, mi, lut):
    c = lax.broadcasted_iota(jnp.int32, (32, 1), 0)
    acc = jnp.broadcast_to(mo[0:1] + dw[0:1] + wd[0:1] + hr[0:1] + mi[0:1], (32, D))
    for f, (ref, r) in enumerate(zip((mo, dw, wd, hr, mi), _MAXROW)):
        bit = ((c >> f) & 1).astype(jnp.float32)
        acc = acc + bit * (ref[r:r + 1] - ref[0:1])
    lut[...] = acc


_build_lut = pl.pallas_call(
    _lut_body,
    out_shape=jax.ShapeDtypeStruct((32, D), jnp.float32),
)


def _sc_body(f0, f1, f2, f3, f4, lut_hbm, out_hbm, x0, x1, x2, x3, x4, codes_v, rows_v, gsem, ssem):
    cid = lax.axis_index("c")
    sid = lax.axis_index("s")
    wid = sid * NC + cid
    b = wid // WPB
    off = (wid % WPB) * SPW

    xbufs = (x0, x1, x2, x3, x4)
    for f, fh in enumerate((f0, f1, f2, f3, f4)):
        pltpu.sync_copy(fh.at[pl.ds(wid * TPW, TPW)], xbufs[f])

    def cbody(j, carry):
        sl = pl.ds(j * L, L)
        code = jnp.zeros((L,), jnp.int32)
        for f in range(F):
            g = xbufs[f][sl]
            code = code | jnp.where(g >= 1, 1 << f, 0)
        codes_v[sl] = code
        return carry

    lax.fori_loop(0, TPW // L, cbody, 0)

    def start_g(i):
        return pltpu.async_copy(
            lut_hbm.at[codes_v.at[pl.ds(i * CHUNK, CHUNK)]],
            rows_v.at[i % NBUF], gsem)

    def start_s(i):
        return pltpu.async_copy(
            rows_v.at[i % NBUF],
            out_hbm.at[b, pl.ds(off + i * CHUNK, CHUNK), :], ssem)

    gcp = {0: start_g(0)}
    scp = {}
    for i in range(NCHUNKS):
        gcp[i].wait()
        if i >= 1:
            scp[i - 1].wait()
        if i + 1 < NCHUNKS:
            gcp[i + 1] = start_g(i + 1)
        scp[i] = start_s(i)
    scp[NCHUNKS - 1].wait()


@functools.cache
def _sc_gather():
    return pl.kernel(
        _sc_body,
        out_type=jax.ShapeDtypeStruct((B, S, D), jnp.float32),
        mesh=plsc.VectorSubcoreMesh(
            core_axis_name="c", subcore_axis_name="s",
            num_cores=NC, num_subcores=NS),
        scratch_types=[
            pltpu.VMEM((TPW,), jnp.int32),
            pltpu.VMEM((TPW,), jnp.int32),
            pltpu.VMEM((TPW,), jnp.int32),
            pltpu.VMEM((TPW,), jnp.int32),
            pltpu.VMEM((TPW,), jnp.int32),
            pltpu.VMEM((TPW,), jnp.int32),
            pltpu.VMEM((NBUF, CHUNK, D), jnp.float32),
            pltpu.SemaphoreType.DMA,
            pltpu.SemaphoreType.DMA,
        ],
    )


def kernel(x, month_w, day_w, weekday_w, hour_w, minute_w):
    lut = _build_lut(month_w, day_w, weekday_w, hour_w, minute_w)
    xi = x.astype(jnp.int32).reshape(N, F)
    feats = tuple(xi[:, f] for f in range(F))
    return _sc_gather()(*feats, lut)


# speedup vs baseline: 8.0077x; 2.1283x over previous
import functools

import jax
import jax.numpy as jnp
from jax import lax
from jax.experimental import pallas as pl
from jax.experimental.pallas import tpu as pltpu
from jax.experimental.pallas import tpu_sc as plsc

D = 1024
B, S, F = 4, 8192, 5
NC, NS, L = 2, 16, 16
NW = NC * NS
N = B * S
TPW = N // NW
CHUNK = 32
NCHUNKS = TPW // CHUNK
NBUF = 2
SPW = S // (NW // B)
WPB = NW // B

_MAXROW = (11, 30, 6, 23, 3)


def _lut_body(mo, dw, wd, hr, mi, lut):
    c = lax.broadcasted_iota(jnp.int32, (32, 1), 0)
    acc = jnp.broadcast_to(mo[0:1] + dw[0:1] + wd[0:1] + hr[0:1] + mi[0:1], (32, D))
    for f, (ref, r) in enumerate(zip((mo, dw, wd, hr, mi), _MAXROW)):
        bit = ((c >> f) & 1).astype(jnp.float32)
        acc = acc + bit * (ref[r:r + 1] - ref[0:1])
    for w in range(NW):
        lut[pl.ds(w * 32, 32), :] = acc


_build_lut = pl.pallas_call(
    _lut_body,
    out_shape=jax.ShapeDtypeStruct((NW * 32, D), jnp.float32),
)


def _sc_body(f0, f1, f2, f3, f4, lut_hbm, out_hbm, x0, x1, x2, x3, x4, codes_v, rows_v, gsem, ssem):
    cid = lax.axis_index("c")
    sid = lax.axis_index("s")
    wid = sid * NC + cid
    b = wid // WPB
    off = (wid % WPB) * SPW

    xbufs = (x0, x1, x2, x3, x4)
    for f, fh in enumerate((f0, f1, f2, f3, f4)):
        pltpu.sync_copy(fh.at[pl.ds(wid * TPW, TPW)], xbufs[f])

    lut_base = wid * 32

    def cbody(j, carry):
        sl = pl.ds(j * L, L)
        code = jnp.broadcast_to(lut_base, (L,))
        for f in range(F):
            g = xbufs[f][sl]
            code = code | jnp.where(g >= 1, 1 << f, 0)
        codes_v[sl] = code
        return carry

    lax.fori_loop(0, TPW // L, cbody, 0)

    def start_g(i):
        return pltpu.async_copy(
            lut_hbm.at[codes_v.at[pl.ds(i * CHUNK, CHUNK)]],
            rows_v.at[i % NBUF], gsem)

    def start_s(i):
        return pltpu.async_copy(
            rows_v.at[i % NBUF],
            out_hbm.at[b, pl.ds(off + i * CHUNK, CHUNK), :], ssem)

    gcp = {0: start_g(0)}
    scp = {}
    for i in range(NCHUNKS):
        gcp[i].wait()
        if i >= 1:
            scp[i - 1].wait()
        if i + 1 < NCHUNKS:
            gcp[i + 1] = start_g(i + 1)
        scp[i] = start_s(i)
    scp[NCHUNKS - 1].wait()


@functools.cache
def _sc_gather():
    return pl.kernel(
        _sc_body,
        out_type=jax.ShapeDtypeStruct((B, S, D), jnp.float32),
        mesh=plsc.VectorSubcoreMesh(
            core_axis_name="c", subcore_axis_name="s",
            num_cores=NC, num_subcores=NS),
        scratch_types=[
            pltpu.VMEM((TPW,), jnp.int32),
            pltpu.VMEM((TPW,), jnp.int32),
            pltpu.VMEM((TPW,), jnp.int32),
            pltpu.VMEM((TPW,), jnp.int32),
            pltpu.VMEM((TPW,), jnp.int32),
            pltpu.VMEM((TPW,), jnp.int32),
            pltpu.VMEM((NBUF, CHUNK, D), jnp.float32),
            pltpu.SemaphoreType.DMA,
            pltpu.SemaphoreType.DMA,
        ],
    )


def kernel(x, month_w, day_w, weekday_w, hour_w, minute_w):
    lut = _build_lut(month_w, day_w, weekday_w, hour_w, minute_w)
    xi = x.astype(jnp.int32).reshape(N, F)
    feats = tuple(xi[:, f] for f in range(F))
    return _sc_gather()(*feats, lut)


# speedup vs baseline: 8.0233x; 1.0019x over previous
import functools

import jax
import jax.numpy as jnp
from jax import lax
from jax.experimental import pallas as pl
from jax.experimental.pallas import tpu as pltpu
from jax.experimental.pallas import tpu_sc as plsc

D = 1024
B, S, F = 4, 8192, 5
NC, NS, L = 2, 16, 16
NW = NC * NS
N = B * S
TPW = N // NW
CHUNK = 32
NCHUNKS = TPW // CHUNK
NBUF = 3
SPW = S // (NW // B)
WPB = NW // B

_MAXROW = (11, 30, 6, 23, 3)


def _lut_body(mo, dw, wd, hr, mi, lut):
    c = lax.broadcasted_iota(jnp.int32, (32, 1), 0)
    acc = jnp.broadcast_to(mo[0:1] + dw[0:1] + wd[0:1] + hr[0:1] + mi[0:1], (32, D))
    for f, (ref, r) in enumerate(zip((mo, dw, wd, hr, mi), _MAXROW)):
        bit = ((c >> f) & 1).astype(jnp.float32)
        acc = acc + bit * (ref[r:r + 1] - ref[0:1])
    for w in range(NW):
        lut[pl.ds(w * 32, 32), :] = acc


_build_lut = pl.pallas_call(
    _lut_body,
    out_shape=jax.ShapeDtypeStruct((NW * 32, D), jnp.float32),
)


def _sc_body(f0, f1, f2, f3, f4, lut_hbm, out_hbm, x0, x1, x2, x3, x4, codes_v, rows_v, gsem, ssem):
    cid = lax.axis_index("c")
    sid = lax.axis_index("s")
    wid = sid * NC + cid
    b = wid // WPB
    off = (wid % WPB) * SPW

    xbufs = (x0, x1, x2, x3, x4)
    for f, fh in enumerate((f0, f1, f2, f3, f4)):
        pltpu.sync_copy(fh.at[pl.ds(wid * TPW, TPW)], xbufs[f])

    lut_base = wid * 32

    def cbody(j, carry):
        sl = pl.ds(j * L, L)
        code = jnp.broadcast_to(lut_base, (L,))
        for f in range(F):
            g = xbufs[f][sl]
            code = code | jnp.where(g >= 1, 1 << f, 0)
        codes_v[sl] = code
        return carry

    lax.fori_loop(0, TPW // L, cbody, 0)

    def start_g(i):
        return pltpu.async_copy(
            lut_hbm.at[codes_v.at[pl.ds(i * CHUNK, CHUNK)]],
            rows_v.at[i % NBUF], gsem)

    def start_s(i):
        return pltpu.async_copy(
            rows_v.at[i % NBUF],
            out_hbm.at[b, pl.ds(off + i * CHUNK, CHUNK), :], ssem)

    gcp = {0: start_g(0)}
    scp = {}
    for i in range(NCHUNKS):
        gcp[i].wait()
        if i + 1 < NCHUNKS:
            if i + 1 - NBUF >= 0:
                scp[i + 1 - NBUF].wait()
            gcp[i + 1] = start_g(i + 1)
        scp[i] = start_s(i)
    for j in range(max(0, NCHUNKS - NBUF), NCHUNKS):
        scp[j].wait()


@functools.cache
def _sc_gather():
    return pl.kernel(
        _sc_body,
        out_type=jax.ShapeDtypeStruct((B, S, D), jnp.float32),
        mesh=plsc.VectorSubcoreMesh(
            core_axis_name="c", subcore_axis_name="s",
            num_cores=NC, num_subcores=NS),
        scratch_types=[
            pltpu.VMEM((TPW,), jnp.int32),
            pltpu.VMEM((TPW,), jnp.int32),
            pltpu.VMEM((TPW,), jnp.int32),
            pltpu.VMEM((TPW,), jnp.int32),
            pltpu.VMEM((TPW,), jnp.int32),
            pltpu.VMEM((TPW,), jnp.int32),
            pltpu.VMEM((NBUF, CHUNK, D), jnp.float32),
            pltpu.SemaphoreType.DMA,
            pltpu.SemaphoreType.DMA,
        ],
    )


def kernel(x, month_w, day_w, weekday_w, hour_w, minute_w):
    lut = _build_lut(month_w, day_w, weekday_w, hour_w, minute_w)
    xi = x.astype(jnp.int32).reshape(N, F)
    feats = tuple(xi[:, f] for f in range(F))
    return _sc_gather()(*feats, lut)


# speedup vs baseline: 8.1740x; 1.0188x over previous
import functools

import jax
import jax.numpy as jnp
from jax import lax
from jax.experimental import pallas as pl
from jax.experimental.pallas import tpu as pltpu
from jax.experimental.pallas import tpu_sc as plsc

D = 1024
B, S, F = 4, 8192, 5
NC, NS, L = 2, 16, 16
NW = NC * NS
N = B * S
TPW = N // NW
HALF = TPW // 2
CHUNK = 16
NCHUNKS = HALF // CHUNK
NBUF = 3
SPW = S // (NW // B)
WPB = NW // B

_MAXROW = (11, 30, 6, 23, 3)


def _lut_body(mo, dw, wd, hr, mi, lut2):
    c = lax.broadcasted_iota(jnp.int32, (32, 1), 0)
    acc = jnp.broadcast_to(mo[0:1] + dw[0:1] + wd[0:1] + hr[0:1] + mi[0:1], (32, D))
    for f, (ref, r) in enumerate(zip((mo, dw, wd, hr, mi), _MAXROW)):
        bit = ((c >> f) & 1).astype(jnp.float32)
        acc = acc + bit * (ref[r:r + 1] - ref[0:1])
    for a in range(32):
        lut2[pl.ds(a * 32, 32), 0:D] = jnp.broadcast_to(acc[a:a + 1], (32, D))
        lut2[pl.ds(a * 32, 32), D:2 * D] = acc


_build_lut = pl.pallas_call(
    _lut_body,
    out_shape=jax.ShapeDtypeStruct((1024, 2 * D), jnp.float32),
)


def _sc_body(f0, f1, f2, f3, f4, lut_hbm, out_hbm, x0, x1, x2, x3, x4,
             codes_v, pair_v, rows_v, gsem, ssem):
    cid = lax.axis_index("c")
    sid = lax.axis_index("s")
    wid = sid * NC + cid
    b = wid // WPB
    off = (wid % WPB) * SPW

    xbufs = (x0, x1, x2, x3, x4)
    for f, fh in enumerate((f0, f1, f2, f3, f4)):
        pltpu.sync_copy(fh.at[pl.ds(wid * TPW, TPW)], xbufs[f])

    def cbody(j, carry):
        sl = pl.ds(j * L, L)
        code = jnp.zeros((L,), jnp.int32)
        for f in range(F):
            g = xbufs[f][sl]
            code = code | jnp.where(g >= 1, 1 << f, 0)
        codes_v[sl] = code
        return carry

    lax.fori_loop(0, TPW // L, cbody, 0)

    def pbody(j, carry):
        sl = pl.ds(j * L, L)
        a = codes_v[sl]
        bcode = codes_v[pl.ds(HALF + j * L, L)]
        pair_v[sl] = a * 32 + bcode
        return carry

    lax.fori_loop(0, HALF // L, pbody, 0)

    def start_g(i):
        return pltpu.async_copy(
            lut_hbm.at[pair_v.at[pl.ds(i * CHUNK, CHUNK)]],
            rows_v.at[i % NBUF], gsem)

    def start_s(i):
        p = i % NBUF
        c0 = pltpu.async_copy(
            rows_v.at[p].at[:, pl.ds(0, D)],
            out_hbm.at[b, pl.ds(off + i * CHUNK, CHUNK), :], ssem)
        c1 = pltpu.async_copy(
            rows_v.at[p].at[:, pl.ds(D, D)],
            out_hbm.at[b, pl.ds(off + HALF + i * CHUNK, CHUNK), :], ssem)
        return (c0, c1)

    gcp = {0: start_g(0)}
    scp = {}
    for i in range(NCHUNKS):
        gcp[i].wait()
        if i + 1 < NCHUNKS:
            if i + 1 - NBUF >= 0:
                for c in scp[i + 1 - NBUF]:
                    c.wait()
            gcp[i + 1] = start_g(i + 1)
        scp[i] = start_s(i)
    for j in range(max(0, NCHUNKS - NBUF), NCHUNKS):
        for c in scp[j]:
            c.wait()


@functools.cache
def _sc_gather():
    return pl.kernel(
        _sc_body,
        out_type=jax.ShapeDtypeStruct((B, S, D), jnp.float32),
        mesh=plsc.VectorSubcoreMesh(
            core_axis_name="c", subcore_axis_name="s",
            num_cores=NC, num_subcores=NS),
        scratch_types=[
            pltpu.VMEM((TPW,), jnp.int32),
            pltpu.VMEM((TPW,), jnp.int32),
            pltpu.VMEM((TPW,), jnp.int32),
            pltpu.VMEM((TPW,), jnp.int32),
            pltpu.VMEM((TPW,), jnp.int32),
            pltpu.VMEM((TPW,), jnp.int32),
            pltpu.VMEM((HALF,), jnp.int32),
            pltpu.VMEM((NBUF, CHUNK, 2 * D), jnp.float32),
            pltpu.SemaphoreType.DMA,
            pltpu.SemaphoreType.DMA,
        ],
    )


def kernel(x, month_w, day_w, weekday_w, hour_w, minute_w):
    lut2 = _build_lut(month_w, day_w, weekday_w, hour_w, minute_w)
    xi = x.astype(jnp.int32).reshape(N, F)
    feats = tuple(xi[:, f] for f in range(F))
    return _sc_gather()(*feats, lut2)


# speedup vs baseline: 8.5649x; 1.0478x over previous
import functools

import jax
import jax.numpy as jnp
from jax import lax
from jax.experimental import pallas as pl
from jax.experimental.pallas import tpu as pltpu
from jax.experimental.pallas import tpu_sc as plsc

D = 1024
B, S, F = 4, 8192, 5
NC, NS, L = 2, 16, 16
NW = NC * NS
N = B * S
TPW = N // NW
HALF = TPW // 2
CHUNK = 16
NCHUNKS = HALF // CHUNK
NBUF = 3
AHEAD = 2
SPW = S // (NW // B)
WPB = NW // B

_MAXROW = (11, 30, 6, 23, 3)


def _lut_body(mo, dw, wd, hr, mi, lut2):
    c = lax.broadcasted_iota(jnp.int32, (32, 1), 0)
    acc = jnp.broadcast_to(mo[0:1] + dw[0:1] + wd[0:1] + hr[0:1] + mi[0:1], (32, D))
    for f, (ref, r) in enumerate(zip((mo, dw, wd, hr, mi), _MAXROW)):
        bit = ((c >> f) & 1).astype(jnp.float32)
        acc = acc + bit * (ref[r:r + 1] - ref[0:1])
    for rep in range(NC):
        for a in range(32):
            base = rep * 1024 + a * 32
            lut2[pl.ds(base, 32), 0:D] = jnp.broadcast_to(acc[a:a + 1], (32, D))
            lut2[pl.ds(base, 32), D:2 * D] = acc


_build_lut = pl.pallas_call(
    _lut_body,
    out_shape=jax.ShapeDtypeStruct((NC * 1024, 2 * D), jnp.float32),
)


def _sc_body(f0, f1, f2, f3, f4, lut_hbm, out_hbm, x0, x1, x2, x3, x4,
             codes_v, pair_v, rows_v, gsem, ssem):
    cid = lax.axis_index("c")
    sid = lax.axis_index("s")
    wid = sid * NC + cid
    b = wid // WPB
    off = (wid % WPB) * SPW

    xbufs = (x0, x1, x2, x3, x4)
    xcps = [pltpu.async_copy(fh.at[pl.ds(wid * TPW, TPW)], xbufs[f], gsem)
            for f, fh in enumerate((f0, f1, f2, f3, f4))]
    for cp in xcps:
        cp.wait()

    def cbody(j, carry):
        sl = pl.ds(j * L, L)
        code = jnp.zeros((L,), jnp.int32)
        for f in range(F):
            g = xbufs[f][sl]
            code = code | jnp.where(g >= 1, 1 << f, 0)
        codes_v[sl] = code
        return carry

    lax.fori_loop(0, TPW // L, cbody, 0)

    lut_rep_base = cid * 1024

    def pbody(j, carry):
        sl = pl.ds(j * L, L)
        a = codes_v[sl]
        bcode = codes_v[pl.ds(HALF + j * L, L)]
        pair_v[sl] = lut_rep_base + a * 32 + bcode
        return carry

    lax.fori_loop(0, HALF // L, pbody, 0)

    def start_g(i):
        return pltpu.async_copy(
            lut_hbm.at[pair_v.at[pl.ds(i * CHUNK, CHUNK)]],
            rows_v.at[i % NBUF], gsem)

    def start_s(i):
        p = i % NBUF
        c0 = pltpu.async_copy(
            rows_v.at[p].at[:, pl.ds(0, D)],
            out_hbm.at[b, pl.ds(off + i * CHUNK, CHUNK), :], ssem)
        c1 = pltpu.async_copy(
            rows_v.at[p].at[:, pl.ds(D, D)],
            out_hbm.at[b, pl.ds(off + HALF + i * CHUNK, CHUNK), :], ssem)
        return (c0, c1)

    gcp = {i: start_g(i) for i in range(min(AHEAD, NCHUNKS))}
    scp = {}
    for i in range(NCHUNKS):
        gcp[i].wait()
        if i + AHEAD < NCHUNKS:
            k = i + AHEAD - NBUF
            if k >= 0:
                for c in scp[k]:
                    c.wait()
            gcp[i + AHEAD] = start_g(i + AHEAD)
        scp[i] = start_s(i)
    for j in range(max(0, NCHUNKS - NBUF), NCHUNKS):
        for c in scp[j]:
            c.wait()


@functools.cache
def _sc_gather():
    return pl.kernel(
        _sc_body,
        out_type=jax.ShapeDtypeStruct((B, S, D), jnp.float32),
        mesh=plsc.VectorSubcoreMesh(
            core_axis_name="c", subcore_axis_name="s",
            num_cores=NC, num_subcores=NS),
        scratch_types=[
            pltpu.VMEM((TPW,), jnp.int32),
            pltpu.VMEM((TPW,), jnp.int32),
            pltpu.VMEM((TPW,), jnp.int32),
            pltpu.VMEM((TPW,), jnp.int32),
            pltpu.VMEM((TPW,), jnp.int32),
            pltpu.VMEM((TPW,), jnp.int32),
            pltpu.VMEM((HALF,), jnp.int32),
            pltpu.VMEM((NBUF, CHUNK, 2 * D), jnp.float32),
            pltpu.SemaphoreType.DMA,
            pltpu.SemaphoreType.DMA,
        ],
    )


def kernel(x, month_w, day_w, weekday_w, hour_w, minute_w):
    lut2 = _build_lut(month_w, day_w, weekday_w, hour_w, minute_w)
    xi = x.astype(jnp.int32).reshape(N, F)
    feats = tuple(xi[:, f] for f in range(F))
    return _sc_gather()(*feats, lut2)
